# baseline (speedup 1.0000x reference)
import functools

import jax
import jax.numpy as jnp
from jax import lax
from jax.experimental import pallas as pl
from jax.experimental.pallas import tpu as pltpu
from jax.experimental.pallas import tpu_sc as plsc

EMBED_DIM = 32
BATCH = 16384
_LANE = 128

_info = plsc.get_sparse_core_info()
_NC, _NS = _info.num_cores, _info.num_subcores
_NW = _NC * _NS
_BPW = BATCH // _NW
_GRP = 4
_NBUF = 4
_NG = _BPW // _GRP

_mesh = plsc.VectorSubcoreMesh(core_axis_name="c", subcore_axis_name="s")


def _extract(vec, k):
    return lax.squeeze(lax.slice(vec, (k,), (k + 1,)), (0,))


@functools.partial(
    pl.kernel,
    mesh=_mesh,
    out_type=jax.ShapeDtypeStruct((EMBED_DIM, BATCH), jnp.float32),
    scratch_types=[
        pltpu.VMEM((_BPW + 16,), jnp.int32),
        pltpu.VMEM((_NBUF, _GRP, EMBED_DIM, _LANE), jnp.float32),
        pltpu.VMEM((EMBED_DIM, _BPW), jnp.float32),
        pltpu.SemaphoreType.DMA,
        pltpu.SemaphoreType.DMA,
        pltpu.SemaphoreType.DMA,
        pltpu.SemaphoreType.DMA,
    ],
    compiler_params=pltpu.CompilerParams(needs_layout_passes=False),
)
def _sc_slab_gather(label_hbm, table_t_hbm, out_t_hbm,
                    lab_v, slabs, out_local, sem0, sem1, sem2, sem3):
    wid = lax.axis_index("s") * _NC + lax.axis_index("c")
    base = wid * _BPW
    pltpu.sync_copy(label_hbm.at[pl.ds(base, _BPW)],
                    lab_v.at[pl.ds(0, _BPW)])

    rows0 = lax.iota(jnp.int32, 16)
    rows1 = rows0 + 16
    sems = (sem0, sem1, sem2, sem3)

    def issue(g, buf):
        labs = lab_v[pl.ds(g * _GRP, 16)]
        for k in range(_GRP):
            i = _extract(labs, k)
            col = pl.multiple_of((i >> 7) * _LANE, _LANE)
            for tr in range(EMBED_DIM // 8):
                pltpu.async_copy(
                    table_t_hbm.at[pl.ds(8 * tr, 8), pl.ds(col, _LANE)],
                    slabs.at[buf, k, pl.ds(8 * tr, 8)],
                    sems[buf],
                )

    def drain(buf):
        for k in range(_GRP):
            pltpu.make_async_copy(
                table_t_hbm.at[:, pl.ds(0, _LANE)],
                slabs.at[buf, k],
                sems[buf],
            ).wait()

    def extract(g, buf):
        labs = lab_v[pl.ds(g * _GRP, 16)]
        for k in range(_GRP):
            i = _extract(labs, k)
            col_l = lax.broadcast(i & (_LANE - 1), (16,))
            col_j = lax.broadcast(g * _GRP + k, (16,))
            slab = slabs.at[buf, k]
            v0 = plsc.load_gather(slab, [rows0, col_l])
            v1 = plsc.load_gather(slab, [rows1, col_l])
            plsc.store_scatter(out_local, [rows0, col_j], v0)
            plsc.store_scatter(out_local, [rows1, col_j], v1)

    for p in range(_NBUF - 1):
        issue(p, p)

    def body(t, carry):
        for p in range(_NBUF):
            g = _NBUF * t + p
            issue(g + _NBUF - 1, (p + _NBUF - 1) % _NBUF)
            drain(p)
            extract(g, p)
        return carry

    lax.fori_loop(0, _NG // _NBUF - 1, body, 0)

    tail = _NG - _NBUF
    for p in range(_NBUF):
        g = tail + p
        if p == 0:
            issue(g + _NBUF - 1, (_NBUF - 1) % _NBUF)
        drain(p)
        extract(g, p)

    pltpu.sync_copy(out_local, out_t_hbm.at[:, pl.ds(base, _BPW)])


def kernel(label, emb_weight):
    out_t = _sc_slab_gather(label.astype(jnp.int32), emb_weight.T)
    return out_t.T


# speedup vs baseline: 1.0076x; 1.0076x over previous
import functools

import jax
import jax.numpy as jnp
from jax import lax
from jax.experimental import pallas as pl
from jax.experimental.pallas import tpu as pltpu
from jax.experimental.pallas import tpu_sc as plsc

EMBED_DIM = 32
BATCH = 16384
_LANE = 128

_info = plsc.get_sparse_core_info()
_NC, _NS = _info.num_cores, _info.num_subcores
_NW = _NC * _NS
_BPW = BATCH // _NW
_GRP = 4
_NBUF = 4
_NG = _BPW // _GRP

_mesh = plsc.VectorSubcoreMesh(core_axis_name="c", subcore_axis_name="s")


def _extract(vec, k):
    return lax.squeeze(lax.slice(vec, (k,), (k + 1,)), (0,))


@functools.partial(
    pl.kernel,
    mesh=_mesh,
    out_type=jax.ShapeDtypeStruct((EMBED_DIM, BATCH), jnp.float32),
    scratch_types=[
        pltpu.VMEM((_BPW + 16,), jnp.int32),
        pltpu.VMEM((_NBUF, _GRP, EMBED_DIM, _LANE), jnp.float32),
        pltpu.VMEM((EMBED_DIM, _BPW), jnp.float32),
        pltpu.SemaphoreType.DMA,
        pltpu.SemaphoreType.DMA,
        pltpu.SemaphoreType.DMA,
        pltpu.SemaphoreType.DMA,
    ],
    compiler_params=pltpu.CompilerParams(needs_layout_passes=False),
)
def _sc_slab_gather(label_hbm, table_t_hbm, out_t_hbm,
                    lab_v, slabs, out_local, sem0, sem1, sem2, sem3):
    wid = lax.axis_index("s") * _NC + lax.axis_index("c")
    base = wid * _BPW
    pltpu.sync_copy(label_hbm.at[pl.ds(base, _BPW)],
                    lab_v.at[pl.ds(0, _BPW)])

    rows0 = lax.iota(jnp.int32, 16)
    rows1 = rows0 + 16
    sems = (sem0, sem1, sem2, sem3)

    def issue(g, buf):
        labs = lab_v[pl.ds(g * _GRP, 16)]
        for k in range(_GRP):
            i = _extract(labs, k)
            col = pl.multiple_of((i >> 7) * _LANE, _LANE)
            pltpu.async_copy(
                table_t_hbm.at[:, pl.ds(col, _LANE)],
                slabs.at[buf, k],
                sems[buf],
            )

    def drain(buf):
        for k in range(_GRP):
            pltpu.make_async_copy(
                table_t_hbm.at[:, pl.ds(0, _LANE)],
                slabs.at[buf, k],
                sems[buf],
            ).wait()

    def extract(g, buf):
        labs = lab_v[pl.ds(g * _GRP, 16)]
        for k in range(_GRP):
            i = _extract(labs, k)
            col_l = lax.broadcast(i & (_LANE - 1), (16,))
            col_j = lax.broadcast(g * _GRP + k, (16,))
            slab = slabs.at[buf, k]
            v0 = plsc.load_gather(slab, [rows0, col_l])
            v1 = plsc.load_gather(slab, [rows1, col_l])
            plsc.store_scatter(out_local, [rows0, col_j], v0)
            plsc.store_scatter(out_local, [rows1, col_j], v1)

    for p in range(_NBUF - 1):
        issue(p, p)

    def body(t, carry):
        for p in range(_NBUF):
            g = _NBUF * t + p
            issue(g + _NBUF - 1, (p + _NBUF - 1) % _NBUF)
            drain(p)
            extract(g, p)
        return carry

    lax.fori_loop(0, _NG // _NBUF - 1, body, 0)

    tail = _NG - _NBUF
    for p in range(_NBUF):
        g = tail + p
        if p == 0:
            issue(g + _NBUF - 1, (_NBUF - 1) % _NBUF)
        drain(p)
        extract(g, p)

    pltpu.sync_copy(out_local, out_t_hbm.at[:, pl.ds(base, _BPW)])


def kernel(label, emb_weight):
    out_t = _sc_slab_gather(label.astype(jnp.int32), emb_weight.T)
    return out_t.T
